# baseline (speedup 1.0000x reference)
import jax
import jax.numpy as jnp
from jax import lax
from jax.experimental import pallas as pl
from jax.experimental.pallas import tpu as pltpu
from jax.experimental.pallas import tpu_sc as plsc

MAX_NUM_TILES = 4
HIDDEN = 1280
TABLE_W = MAX_NUM_TILES * HIDDEN
NUM_PATCHES = 1601
BATCH = 32
NLANE = 16
NVH = HIDDEN // NLANE
PCH = 67


def _sc_gather_body(ar_hbm, ti_hbm, table_hbm, gate_hbm, out_hbm,
                    ar_v, ti_v, idx_v, rows_v, srow_v, gate_v, sem):
    c = lax.axis_index("c")
    s = lax.axis_index("s")
    num_c = lax.axis_size("c")
    wid = s * num_c + c

    @pl.when(wid < 2)
    def _():
        pltpu.sync_copy(ar_hbm, ar_v)
        pltpu.sync_copy(ti_hbm, ti_v)
        pltpu.sync_copy(gate_hbm, gate_v)
        base = wid * NLANE
        idx_v[...] = ar_v[pl.ds(base, NLANE)]
        cp = pltpu.async_copy(table_hbm.at[idx_v], rows_v, sem)
        ti16 = ti_v[pl.ds(base, NLANE)]
        g = gate_v[...]
        scale = 1.0 - 2.0 / (jnp.exp(2.0 * g) + 1.0)
        cp.wait()
        for i in range(NLANE):
            col = ti16[i] * HIDDEN

            def mv(j, carry, i=i, col=col):
                srow_v[i, pl.ds(j * NLANE, NLANE)] = (
                    rows_v[i, pl.ds(col + j * NLANE, NLANE)] * scale)
                return carry
            lax.fori_loop(0, NVH, mv, 0)
        pltpu.sync_copy(srow_v, out_hbm.at[pl.ds(base, NLANE)])


def _sc_gather(ar, ti, table, gate16):
    mesh = plsc.VectorSubcoreMesh(core_axis_name="c", subcore_axis_name="s")
    return pl.kernel(
        _sc_gather_body,
        out_type=jax.ShapeDtypeStruct((BATCH, HIDDEN), jnp.float32),
        mesh=mesh,
        scratch_types=[
            pltpu.VMEM((BATCH,), jnp.int32),
            pltpu.VMEM((BATCH,), jnp.int32),
            pltpu.VMEM((NLANE,), jnp.int32),
            pltpu.VMEM((NLANE, TABLE_W), jnp.float32),
            pltpu.VMEM((NLANE, HIDDEN), jnp.float32),
            pltpu.VMEM((NLANE,), jnp.float32),
            pltpu.SemaphoreType.DMA,
        ],
    )(ar, ti, table, gate16)


def _add_body(xt_ref, emb_ref, o_ref):
    o_ref[...] = xt_ref[...] + emb_ref[...][None]


def _tc_add_t(xt, emb):
    npb = pl.cdiv(NUM_PATCHES, PCH)
    return pl.pallas_call(
        _add_body,
        grid=(npb,),
        in_specs=[
            pl.BlockSpec((PCH, BATCH, HIDDEN), lambda p: (p, 0, 0)),
            pl.BlockSpec((BATCH, HIDDEN), lambda p: (0, 0)),
        ],
        out_specs=pl.BlockSpec((PCH, BATCH, HIDDEN), lambda p: (p, 0, 0)),
        out_shape=jax.ShapeDtypeStruct(xt.shape, xt.dtype),
        compiler_params=pltpu.CompilerParams(
            dimension_semantics=("arbitrary",)),
    )(xt, emb)


@jax.jit
def kernel(x, aspect_ratio_ids, tile_indices, table, gate):
    xt = x.transpose(1, 0, 2)
    gate16 = jnp.broadcast_to(gate.reshape(1), (NLANE,))
    emb = _sc_gather(aspect_ratio_ids.astype(jnp.int32),
                     tile_indices.astype(jnp.int32), table, gate16)
    out_t = _tc_add_t(xt, emb)
    return out_t.transpose(1, 0, 2)


# speedup vs baseline: 1.0568x; 1.0568x over previous
import jax
import jax.numpy as jnp
from jax import lax
from jax.experimental import pallas as pl
from jax.experimental.pallas import tpu as pltpu
from jax.experimental.pallas import tpu_sc as plsc

MAX_NUM_TILES = 4
HIDDEN = 1280
NUM_PATCHES = 1601
BATCH = 32
PCH = 67


def _sc_gather_body(ar_hbm, ti_hbm, table_hbm, out_hbm, ar_v, ti_v, idx_v,
                    rows_v, sem):
    c = lax.axis_index("c")
    s = lax.axis_index("s")
    num_c = lax.axis_size("c")
    wid = s * num_c + c

    @pl.when(wid < 2)
    def _():
        pltpu.sync_copy(ar_hbm, ar_v)
        pltpu.sync_copy(ti_hbm, ti_v)
        base = wid * 16
        ar16 = ar_v[pl.ds(base, 16)]
        ti16 = ti_v[pl.ds(base, 16)]
        idx_v[...] = ar16 * MAX_NUM_TILES + ti16
        pltpu.async_copy(table_hbm.at[idx_v], rows_v, sem).wait()
        pltpu.sync_copy(rows_v, out_hbm.at[pl.ds(base, 16)])


def _sc_gather(ar, ti, table_rows):
    b = ar.shape[0]
    mesh = plsc.VectorSubcoreMesh(core_axis_name="c", subcore_axis_name="s")
    return pl.kernel(
        _sc_gather_body,
        out_type=jax.ShapeDtypeStruct((b, HIDDEN), jnp.float32),
        mesh=mesh,
        scratch_types=[
            pltpu.VMEM((b,), jnp.int32),
            pltpu.VMEM((b,), jnp.int32),
            pltpu.VMEM((16,), jnp.int32),
            pltpu.VMEM((16, HIDDEN), jnp.float32),
            pltpu.SemaphoreType.DMA,
        ],
    )(ar, ti, table_rows)


def _add_body(xt_ref, emb_ref, gate_ref, o_ref):
    scale = jnp.tanh(gate_ref[...])
    o_ref[...] = xt_ref[...] + (emb_ref[...] * scale)[None]


def _tc_add_t(xt, emb, gate2):
    npb = pl.cdiv(NUM_PATCHES, PCH)
    return pl.pallas_call(
        _add_body,
        grid=(npb,),
        in_specs=[
            pl.BlockSpec((PCH, BATCH, HIDDEN), lambda p: (p, 0, 0)),
            pl.BlockSpec((BATCH, HIDDEN), lambda p: (0, 0)),
            pl.BlockSpec((1, 1), lambda p: (0, 0)),
        ],
        out_specs=pl.BlockSpec((PCH, BATCH, HIDDEN), lambda p: (p, 0, 0)),
        out_shape=jax.ShapeDtypeStruct(xt.shape, xt.dtype),
        compiler_params=pltpu.CompilerParams(
            dimension_semantics=("arbitrary",)),
    )(xt, emb, gate2)


@jax.jit
def kernel(x, aspect_ratio_ids, tile_indices, table, gate):
    xt = x.transpose(1, 0, 2)
    table_rows = table.reshape(-1, HIDDEN)
    emb = _sc_gather(aspect_ratio_ids.astype(jnp.int32),
                     tile_indices.astype(jnp.int32), table_rows)
    out_t = _tc_add_t(xt, emb, gate.reshape(1, 1))
    return out_t.transpose(1, 0, 2)
